# baseline (speedup 1.0000x reference)
import functools

import jax
import jax.numpy as jnp
from jax import lax
from jax.experimental import pallas as pl
from jax.experimental.pallas import tpu as pltpu
from jax.experimental.pallas import tpu_sc as plsc

T, A, D = 32, 32, 2048
E, H, NA, ADD = 8, 2048, 32, 12
N = T * A
TM = 128
G = 15
CAP = 2048

NC, NS = 2, 16
NW = NC * NS
OUT_W = 128


def _sc_gather_rows(table, idx, rows_per_worker, chunk):
    B = idx.shape[0]
    Dc = table.shape[1]
    nch = rows_per_worker // chunk
    mesh = plsc.VectorSubcoreMesh(core_axis_name="c", subcore_axis_name="s")

    @functools.partial(
        pl.kernel,
        mesh=mesh,
        out_type=jax.ShapeDtypeStruct((B, Dc), table.dtype),
        scratch_types=[
            pltpu.VMEM((rows_per_worker,), jnp.int32),
            pltpu.VMEM((chunk, Dc), table.dtype),
            pltpu.VMEM((chunk, Dc), table.dtype),
            pltpu.SemaphoreType.DMA,
            pltpu.SemaphoreType.DMA,
            pltpu.SemaphoreType.DMA,
            pltpu.SemaphoreType.DMA,
        ],
    )
    def k(table_hbm, idx_hbm, out_hbm, idx_v, buf_a, buf_b, gs_a, gs_b,
          ss_a, ss_b):
        wid = lax.axis_index("s") * NC + lax.axis_index("c")
        base = wid * rows_per_worker
        pltpu.sync_copy(idx_hbm.at[pl.ds(base, rows_per_worker)], idx_v)
        bufs, gsems, ssems = [buf_a, buf_b], [gs_a, gs_b], [ss_a, ss_b]
        gathers = [None] * nch
        stores = [None] * nch
        gathers[0] = pltpu.async_copy(
            table_hbm.at[idx_v.at[pl.ds(0, chunk)]], bufs[0], gsems[0])
        for c in range(nch):
            b = c % 2
            gathers[c].wait()
            if c + 1 < nch:
                if c >= 1:
                    stores[c - 1].wait()
                gathers[c + 1] = pltpu.async_copy(
                    table_hbm.at[idx_v.at[pl.ds((c + 1) * chunk, chunk)]],
                    bufs[(c + 1) % 2], gsems[(c + 1) % 2])
            stores[c] = pltpu.async_copy(
                bufs[b], out_hbm.at[pl.ds(base + c * chunk, chunk)], ssems[b])
        stores[nch - 1].wait()
        if nch >= 2:
            stores[nch - 2].wait()

    return k(table, idx)


def _tc_expert_tiles(te, valid, xs, W1, W2, b1, b2, ph2f):
    def body(te_ref, valid_ref, x_ref, w1_ref, w2_ref, b1_ref, b2_ref,
             ph2f_ref, out_ref):
        w = pl.program_id(0)
        e = te_ref[w]

        @pl.when(valid_ref[w] == 1)
        def _():
            b1eff = b1_ref[0, 0]
            for a in range(ADD):
                b1eff = b1eff + ph2f_ref[e, a] * w1_ref[0, D + a, :]
            x_bf = x_ref[...].astype(jnp.bfloat16)
            w1a = w1_ref[0, :D, :].astype(jnp.bfloat16)
            h = jnp.dot(x_bf, w1a, preferred_element_type=jnp.float32)
            h = jnp.maximum(h + b1eff[None, :], 0.0)
            w2 = w2_ref[0].astype(jnp.bfloat16)
            logits = jnp.dot(h.astype(jnp.bfloat16), w2,
                             preferred_element_type=jnp.float32)
            logits = logits + b2_ref[0, 0][None, :]
            m = jnp.max(logits, axis=1, keepdims=True)
            lse = jnp.log(jnp.sum(jnp.exp(logits - m), axis=1, keepdims=True))
            out_ref[:, NA:] = jnp.zeros((TM, OUT_W - NA), jnp.float32)
            out_ref[:, :NA] = logits - (m + lse)

        @pl.when(valid_ref[w] == 0)
        def _():
            out_ref[...] = jnp.zeros_like(out_ref)

    grid_spec = pltpu.PrefetchScalarGridSpec(
        num_scalar_prefetch=2,
        grid=(G,),
        in_specs=[
            pl.BlockSpec((TM, D), lambda w, te, v: (w, 0)),
            pl.BlockSpec((1, D + ADD, H), lambda w, te, v: (te[w], 0, 0)),
            pl.BlockSpec((1, H, NA), lambda w, te, v: (te[w], 0, 0)),
            pl.BlockSpec((1, 1, H), lambda w, te, v: (te[w], 0, 0)),
            pl.BlockSpec((1, 1, NA), lambda w, te, v: (te[w], 0, 0)),
            pl.BlockSpec(memory_space=pltpu.SMEM),
        ],
        out_specs=pl.BlockSpec((TM, OUT_W), lambda w, te, v: (w, 0)),
    )
    return pl.pallas_call(
        body,
        grid_spec=grid_spec,
        out_shape=jax.ShapeDtypeStruct((G * TM, OUT_W), jnp.float32),
        compiler_params=pltpu.CompilerParams(
            dimension_semantics=("arbitrary",),
        ),
    )(te, valid, xs, W1, W2, b1.reshape(E, 1, H), b2.reshape(E, 1, NA), ph2f)


def kernel(obs, expert_ids, ph_to_feature, W1, b1, W2, b2):
    x = obs.reshape(N, D)
    eid = expert_ids.reshape(-1).astype(jnp.int32)

    onehot = (eid[:, None] == jnp.arange(E, dtype=jnp.int32)[None, :])
    onehot = onehot.astype(jnp.int32)
    counts = jnp.sum(onehot, axis=0)
    rank = jnp.take_along_axis(jnp.cumsum(onehot, axis=0) - onehot,
                               eid[:, None], axis=1)[:, 0]
    tiles_per_e = (counts + TM - 1) // TM
    ctiles = jnp.cumsum(tiles_per_e)
    tile_start_e = ctiles - tiles_per_e
    pos = tile_start_e[eid] * TM + rank
    gather_idx = jnp.zeros((CAP,), jnp.int32).at[pos].set(
        jnp.arange(N, dtype=jnp.int32))
    total_tiles = ctiles[E - 1]
    t_arr = jnp.arange(G, dtype=jnp.int32)
    te_raw = jnp.searchsorted(ctiles, t_arr, side="right").astype(jnp.int32)
    valid = (t_arr < total_tiles).astype(jnp.int32)
    last_e = jnp.searchsorted(ctiles, total_tiles - 1,
                              side="right").astype(jnp.int32)
    te = jnp.where(valid == 1, jnp.minimum(te_raw, E - 1), last_e)

    xs = _sc_gather_rows(x, gather_idx, rows_per_worker=CAP // NW, chunk=16)

    out_sorted = _tc_expert_tiles(te, valid, xs, W1, W2, b1, b2,
                                  ph_to_feature)

    logp = _sc_gather_rows(out_sorted, pos.astype(jnp.int32),
                           rows_per_worker=N // NW, chunk=N // NW)
    return logp[:, :NA].reshape(T, A, NA)


# speedup vs baseline: 1.0140x; 1.0140x over previous
import functools

import jax
import jax.numpy as jnp
from jax import lax
from jax.experimental import pallas as pl
from jax.experimental.pallas import tpu as pltpu
from jax.experimental.pallas import tpu_sc as plsc

T, A, D = 32, 32, 2048
E, H, NA, ADD = 8, 2048, 32, 12
N = T * A
TM = 128
G = 15
CAP = 2048

NC, NS = 2, 16
NW = NC * NS
OUT_W = 128
KC = D // 128


def _sc_gather_rows(table, idx, rows_per_worker, chunk):
    B = idx.shape[0]
    row_shape = table.shape[1:]
    nch = rows_per_worker // chunk
    mesh = plsc.VectorSubcoreMesh(core_axis_name="c", subcore_axis_name="s")

    @functools.partial(
        pl.kernel,
        mesh=mesh,
        out_type=jax.ShapeDtypeStruct((B,) + row_shape, table.dtype),
        scratch_types=[
            pltpu.VMEM((rows_per_worker,), jnp.int32),
            pltpu.VMEM((chunk,) + row_shape, table.dtype),
            pltpu.VMEM((chunk,) + row_shape, table.dtype),
            pltpu.SemaphoreType.DMA,
            pltpu.SemaphoreType.DMA,
            pltpu.SemaphoreType.DMA,
            pltpu.SemaphoreType.DMA,
        ],
    )
    def k(table_hbm, idx_hbm, out_hbm, idx_v, buf_a, buf_b, gs_a, gs_b,
          ss_a, ss_b):
        wid = lax.axis_index("s") * NC + lax.axis_index("c")
        base = wid * rows_per_worker
        pltpu.sync_copy(idx_hbm.at[pl.ds(base, rows_per_worker)], idx_v)
        bufs, gsems, ssems = [buf_a, buf_b], [gs_a, gs_b], [ss_a, ss_b]
        gathers = [None] * nch
        stores = [None] * nch
        gathers[0] = pltpu.async_copy(
            table_hbm.at[idx_v.at[pl.ds(0, chunk)]], bufs[0], gsems[0])
        for c in range(nch):
            b = c % 2
            gathers[c].wait()
            if c + 1 < nch:
                if c >= 1:
                    stores[c - 1].wait()
                gathers[c + 1] = pltpu.async_copy(
                    table_hbm.at[idx_v.at[pl.ds((c + 1) * chunk, chunk)]],
                    bufs[(c + 1) % 2], gsems[(c + 1) % 2])
            stores[c] = pltpu.async_copy(
                bufs[b], out_hbm.at[pl.ds(base + c * chunk, chunk)], ssems[b])
        stores[nch - 1].wait()
        if nch >= 2:
            stores[nch - 2].wait()

    return k(table, idx)


def _tc_expert_tiles(te, valid, xs, W1, W2, b1, b2, ph2f):
    def body(te_ref, valid_ref, x_ref, w1_ref, w2_ref, b1_ref, b2_ref,
             ph2f_ref, out_ref):
        w = pl.program_id(0)
        e = te_ref[w]

        @pl.when(valid_ref[w] == 1)
        def _():
            b1eff = b1_ref[0, 0]
            for a in range(ADD):
                b1eff = b1eff + ph2f_ref[e, a] * w1_ref[0, D + a, :]
            h = jnp.broadcast_to(b1eff[None, :], (TM, H))
            for j in range(KC):
                x_bf = x_ref[:, j, :].astype(jnp.bfloat16)
                w1a_j = w1_ref[0, j * 128:(j + 1) * 128, :].astype(jnp.bfloat16)
                h = h + jnp.dot(x_bf, w1a_j,
                                preferred_element_type=jnp.float32)
            h = jnp.maximum(h, 0.0)
            w2 = w2_ref[0].astype(jnp.bfloat16)
            logits = jnp.dot(h.astype(jnp.bfloat16), w2,
                             preferred_element_type=jnp.float32)
            logits = logits + b2_ref[0, 0][None, :]
            m = jnp.max(logits, axis=1, keepdims=True)
            lse = jnp.log(jnp.sum(jnp.exp(logits - m), axis=1, keepdims=True))
            out_ref[:, NA:] = jnp.zeros((TM, OUT_W - NA), jnp.float32)
            out_ref[:, :NA] = logits - (m + lse)

        @pl.when(valid_ref[w] == 0)
        def _():
            out_ref[...] = jnp.zeros_like(out_ref)

    grid_spec = pltpu.PrefetchScalarGridSpec(
        num_scalar_prefetch=2,
        grid=(G,),
        in_specs=[
            pl.BlockSpec((TM, KC, 128), lambda w, te, v: (w, 0, 0)),
            pl.BlockSpec((1, D + ADD, H), lambda w, te, v: (te[w], 0, 0)),
            pl.BlockSpec((1, H, NA), lambda w, te, v: (te[w], 0, 0)),
            pl.BlockSpec((1, 1, H), lambda w, te, v: (te[w], 0, 0)),
            pl.BlockSpec((1, 1, NA), lambda w, te, v: (te[w], 0, 0)),
            pl.BlockSpec(memory_space=pltpu.SMEM),
        ],
        out_specs=pl.BlockSpec((TM, OUT_W), lambda w, te, v: (w, 0)),
    )
    return pl.pallas_call(
        body,
        grid_spec=grid_spec,
        out_shape=jax.ShapeDtypeStruct((G * TM, OUT_W), jnp.float32),
        compiler_params=pltpu.CompilerParams(
            dimension_semantics=("arbitrary",),
        ),
    )(te, valid, xs, W1, W2, b1.reshape(E, 1, H), b2.reshape(E, 1, NA), ph2f)


def kernel(obs, expert_ids, ph_to_feature, W1, b1, W2, b2):
    x = obs.reshape(N, KC, 128)
    eid = expert_ids.reshape(-1).astype(jnp.int32)

    onehot = (eid[:, None] == jnp.arange(E, dtype=jnp.int32)[None, :])
    onehot = onehot.astype(jnp.int32)
    counts = jnp.sum(onehot, axis=0)
    rank = jnp.take_along_axis(jnp.cumsum(onehot, axis=0) - onehot,
                               eid[:, None], axis=1)[:, 0]
    tiles_per_e = (counts + TM - 1) // TM
    ctiles = jnp.cumsum(tiles_per_e)
    tile_start_e = ctiles - tiles_per_e
    pos = tile_start_e[eid] * TM + rank
    gather_idx = jnp.zeros((CAP,), jnp.int32).at[pos].set(
        jnp.arange(N, dtype=jnp.int32))
    total_tiles = ctiles[E - 1]
    t_arr = jnp.arange(G, dtype=jnp.int32)
    te_raw = jnp.searchsorted(ctiles, t_arr, side="right").astype(jnp.int32)
    valid = (t_arr < total_tiles).astype(jnp.int32)
    last_e = jnp.searchsorted(ctiles, total_tiles - 1,
                              side="right").astype(jnp.int32)
    te = jnp.where(valid == 1, jnp.minimum(te_raw, E - 1), last_e)

    xs = _sc_gather_rows(x, gather_idx, rows_per_worker=CAP // NW, chunk=16)

    out_sorted = _tc_expert_tiles(te, valid, xs, W1, W2, b1, b2,
                                  ph_to_feature)

    logp = _sc_gather_rows(out_sorted, pos.astype(jnp.int32),
                           rows_per_worker=N // NW, chunk=N // NW)
    return logp[:, :NA].reshape(T, A, NA)


# speedup vs baseline: 1.3039x; 1.2859x over previous
import functools

import jax
import jax.numpy as jnp
from jax import lax
from jax.experimental import pallas as pl
from jax.experimental.pallas import tpu as pltpu
from jax.experimental.pallas import tpu_sc as plsc

T, A, D = 32, 32, 2048
E, H, NA, ADD = 8, 2048, 32, 12
N = T * A
TM = 128
G = 15
CAP = 2048

NC, NS = 2, 16
NW = NC * NS
OUT_W = 128
KC = D // 128


def _sc_gather_rows(table, idx, rows_per_worker, chunk):
    B = idx.shape[0]
    row_shape = table.shape[1:]
    nch = rows_per_worker // chunk
    mesh = plsc.VectorSubcoreMesh(core_axis_name="c", subcore_axis_name="s")

    @functools.partial(
        pl.kernel,
        mesh=mesh,
        out_type=jax.ShapeDtypeStruct((B,) + row_shape, table.dtype),
        scratch_types=[
            pltpu.VMEM((rows_per_worker,), jnp.int32),
            pltpu.VMEM((chunk,) + row_shape, table.dtype),
            pltpu.VMEM((chunk,) + row_shape, table.dtype),
            pltpu.SemaphoreType.DMA,
            pltpu.SemaphoreType.DMA,
            pltpu.SemaphoreType.DMA,
            pltpu.SemaphoreType.DMA,
        ],
    )
    def k(table_hbm, idx_hbm, out_hbm, idx_v, buf_a, buf_b, gs_a, gs_b,
          ss_a, ss_b):
        wid = lax.axis_index("s") * NC + lax.axis_index("c")
        base = wid * rows_per_worker
        pltpu.sync_copy(idx_hbm.at[pl.ds(base, rows_per_worker)], idx_v)
        bufs, gsems, ssems = [buf_a, buf_b], [gs_a, gs_b], [ss_a, ss_b]
        gathers = [None] * nch
        stores = [None] * nch
        gathers[0] = pltpu.async_copy(
            table_hbm.at[idx_v.at[pl.ds(0, chunk)]], bufs[0], gsems[0])
        for c in range(nch):
            b = c % 2
            gathers[c].wait()
            if c + 1 < nch:
                if c >= 1:
                    stores[c - 1].wait()
                gathers[c + 1] = pltpu.async_copy(
                    table_hbm.at[idx_v.at[pl.ds((c + 1) * chunk, chunk)]],
                    bufs[(c + 1) % 2], gsems[(c + 1) % 2])
            stores[c] = pltpu.async_copy(
                bufs[b], out_hbm.at[pl.ds(base + c * chunk, chunk)], ssems[b])
        stores[nch - 1].wait()
        if nch >= 2:
            stores[nch - 2].wait()

    return k(table, idx)


def _tc_expert_tiles(te, valid, pos, x_bf, W1, W2, b1, b2, ph2f):
    def body(te_ref, valid_ref, pos_ref, x_ref, w1_ref, w2_ref, b1_ref,
             b2_ref, ph2f_ref, out_ref):
        w = pl.program_id(0)
        e = te_ref[w]

        @pl.when(valid_ref[w] == 1)
        def _():
            row_ids = jax.lax.broadcasted_iota(jnp.int32, (TM, N), 0) + w * TM
            mask = (row_ids == pos_ref[0][None, :]).astype(jnp.bfloat16)
            x_tile = jnp.dot(mask, x_ref[...],
                             preferred_element_type=jnp.float32)
            x_tile = x_tile.astype(jnp.bfloat16)
            b1eff = b1_ref[0, 0]
            for a in range(ADD):
                b1eff = b1eff + ph2f_ref[e, a] * w1_ref[0, D + a, :]
            w1a = w1_ref[0, :D, :].astype(jnp.bfloat16)
            h = jnp.dot(x_tile, w1a, preferred_element_type=jnp.float32)
            h = jnp.maximum(h + b1eff[None, :], 0.0)
            w2 = w2_ref[0].astype(jnp.bfloat16)
            logits = jnp.dot(h.astype(jnp.bfloat16), w2,
                             preferred_element_type=jnp.float32)
            logits = logits + b2_ref[0, 0][None, :]
            m = jnp.max(logits, axis=1, keepdims=True)
            lse = jnp.log(jnp.sum(jnp.exp(logits - m), axis=1, keepdims=True))
            out_ref[:, NA:] = jnp.zeros((TM, OUT_W - NA), jnp.float32)
            out_ref[:, :NA] = logits - (m + lse)

        @pl.when(valid_ref[w] == 0)
        def _():
            out_ref[...] = jnp.zeros_like(out_ref)

    grid_spec = pltpu.PrefetchScalarGridSpec(
        num_scalar_prefetch=2,
        grid=(G,),
        in_specs=[
            pl.BlockSpec((1, N), lambda w, te, v: (0, 0)),
            pl.BlockSpec((N, D), lambda w, te, v: (0, 0)),
            pl.BlockSpec((1, D + ADD, H), lambda w, te, v: (te[w], 0, 0)),
            pl.BlockSpec((1, H, NA), lambda w, te, v: (te[w], 0, 0)),
            pl.BlockSpec((1, 1, H), lambda w, te, v: (te[w], 0, 0)),
            pl.BlockSpec((1, 1, NA), lambda w, te, v: (te[w], 0, 0)),
            pl.BlockSpec(memory_space=pltpu.SMEM),
        ],
        out_specs=pl.BlockSpec((TM, OUT_W), lambda w, te, v: (w, 0)),
    )
    return pl.pallas_call(
        body,
        grid_spec=grid_spec,
        out_shape=jax.ShapeDtypeStruct((G * TM, OUT_W), jnp.float32),
        compiler_params=pltpu.CompilerParams(
            dimension_semantics=("arbitrary",),
        ),
    )(te, valid, pos, x_bf, W1, W2, b1.reshape(E, 1, H),
      b2.reshape(E, 1, NA), ph2f)


def kernel(obs, expert_ids, ph_to_feature, W1, b1, W2, b2):
    x_bf = obs.reshape(N, D).astype(jnp.bfloat16)
    eid = expert_ids.reshape(-1).astype(jnp.int32)

    onehot = (eid[:, None] == jnp.arange(E, dtype=jnp.int32)[None, :])
    onehot = onehot.astype(jnp.int32)
    counts = jnp.sum(onehot, axis=0)
    rank = jnp.take_along_axis(jnp.cumsum(onehot, axis=0) - onehot,
                               eid[:, None], axis=1)[:, 0]
    tiles_per_e = (counts + TM - 1) // TM
    ctiles = jnp.cumsum(tiles_per_e)
    tile_start_e = ctiles - tiles_per_e
    pos = tile_start_e[eid] * TM + rank
    total_tiles = ctiles[E - 1]
    t_arr = jnp.arange(G, dtype=jnp.int32)
    te_raw = jnp.searchsorted(ctiles, t_arr, side="right").astype(jnp.int32)
    valid = (t_arr < total_tiles).astype(jnp.int32)
    last_e = jnp.searchsorted(ctiles, total_tiles - 1,
                              side="right").astype(jnp.int32)
    te = jnp.where(valid == 1, jnp.minimum(te_raw, E - 1), last_e)

    out_sorted = _tc_expert_tiles(te, valid, pos.reshape(1, N), x_bf,
                                  W1, W2, b1, b2, ph_to_feature)

    logp = _sc_gather_rows(out_sorted, pos.astype(jnp.int32),
                           rows_per_worker=N // NW, chunk=N // NW)
    return logp[:, :NA].reshape(T, A, NA)


# speedup vs baseline: 1.3044x; 1.0004x over previous
import functools

import jax
import jax.numpy as jnp
from jax import lax
from jax.experimental import pallas as pl
from jax.experimental.pallas import tpu as pltpu
from jax.experimental.pallas import tpu_sc as plsc

T, A, D = 32, 32, 2048
E, H, NA, ADD = 8, 2048, 32, 12
N = T * A
TM = 128
G = 15
CAP = 2048

NC, NS = 2, 16
NW = NC * NS
OUT_W = 128
KC = D // 128


def _sc_gather_rows(table, idx, rows_per_worker, chunk):
    B = idx.shape[0]
    row_shape = table.shape[1:]
    nch = rows_per_worker // chunk
    mesh = plsc.VectorSubcoreMesh(core_axis_name="c", subcore_axis_name="s")

    @functools.partial(
        pl.kernel,
        mesh=mesh,
        out_type=jax.ShapeDtypeStruct((B,) + row_shape, table.dtype),
        scratch_types=[
            pltpu.VMEM((rows_per_worker,), jnp.int32),
            pltpu.VMEM((chunk,) + row_shape, table.dtype),
            pltpu.VMEM((chunk,) + row_shape, table.dtype),
            pltpu.SemaphoreType.DMA,
            pltpu.SemaphoreType.DMA,
            pltpu.SemaphoreType.DMA,
            pltpu.SemaphoreType.DMA,
        ],
    )
    def k(table_hbm, idx_hbm, out_hbm, idx_v, buf_a, buf_b, gs_a, gs_b,
          ss_a, ss_b):
        wid = lax.axis_index("s") * NC + lax.axis_index("c")
        base = wid * rows_per_worker
        pltpu.sync_copy(idx_hbm.at[pl.ds(base, rows_per_worker)], idx_v)
        bufs, gsems, ssems = [buf_a, buf_b], [gs_a, gs_b], [ss_a, ss_b]
        gathers = [None] * nch
        stores = [None] * nch
        gathers[0] = pltpu.async_copy(
            table_hbm.at[idx_v.at[pl.ds(0, chunk)]], bufs[0], gsems[0])
        for c in range(nch):
            b = c % 2
            gathers[c].wait()
            if c + 1 < nch:
                if c >= 1:
                    stores[c - 1].wait()
                gathers[c + 1] = pltpu.async_copy(
                    table_hbm.at[idx_v.at[pl.ds((c + 1) * chunk, chunk)]],
                    bufs[(c + 1) % 2], gsems[(c + 1) % 2])
            stores[c] = pltpu.async_copy(
                bufs[b], out_hbm.at[pl.ds(base + c * chunk, chunk)], ssems[b])
        stores[nch - 1].wait()
        if nch >= 2:
            stores[nch - 2].wait()

    return k(table, idx)


def _tc_expert_tiles(te, valid, pos, x_bf, W1, W2, b1, b2, ph2f):
    def body(te_ref, valid_ref, pos_ref, x_ref, w1_ref, w2_ref, b1_ref,
             b2_ref, ph2f_ref, out_ref):
        w = pl.program_id(0)
        e = te_ref[w]

        @pl.when(valid_ref[w] == 1)
        def _():
            row_ids = jax.lax.broadcasted_iota(jnp.int32, (TM, N), 0) + w * TM
            mask = (row_ids == pos_ref[0][None, :]).astype(jnp.bfloat16)
            x_tile = jnp.dot(mask, x_ref[...],
                             preferred_element_type=jnp.float32)
            x_tile = x_tile.astype(jnp.bfloat16)
            b1eff = b1_ref[0, 0]
            for a in range(ADD):
                b1eff = b1eff + ph2f_ref[e, a] * w1_ref[0, D + a, :]
            h = jnp.dot(x_tile, w1_ref[0, :D, :],
                        preferred_element_type=jnp.float32)
            h = jnp.maximum(h + b1eff[None, :], 0.0)
            logits = jnp.dot(h.astype(jnp.bfloat16), w2_ref[0],
                             preferred_element_type=jnp.float32)
            logits = logits + b2_ref[0, 0][None, :]
            m = jnp.max(logits, axis=1, keepdims=True)
            lse = jnp.log(jnp.sum(jnp.exp(logits - m), axis=1, keepdims=True))
            out_ref[:, NA:] = jnp.zeros((TM, OUT_W - NA), jnp.float32)
            out_ref[:, :NA] = logits - (m + lse)

        @pl.when(valid_ref[w] == 0)
        def _():
            out_ref[...] = jnp.zeros_like(out_ref)

    grid_spec = pltpu.PrefetchScalarGridSpec(
        num_scalar_prefetch=2,
        grid=(G,),
        in_specs=[
            pl.BlockSpec((1, N), lambda w, te, v: (0, 0)),
            pl.BlockSpec((N, D), lambda w, te, v: (0, 0)),
            pl.BlockSpec((1, D + ADD, H), lambda w, te, v: (te[w], 0, 0)),
            pl.BlockSpec((1, H, NA), lambda w, te, v: (te[w], 0, 0)),
            pl.BlockSpec((1, 1, H), lambda w, te, v: (te[w], 0, 0)),
            pl.BlockSpec((1, 1, NA), lambda w, te, v: (te[w], 0, 0)),
            pl.BlockSpec(memory_space=pltpu.SMEM),
        ],
        out_specs=pl.BlockSpec((TM, OUT_W), lambda w, te, v: (w, 0)),
    )
    return pl.pallas_call(
        body,
        grid_spec=grid_spec,
        out_shape=jax.ShapeDtypeStruct((G * TM, OUT_W), jnp.float32),
        compiler_params=pltpu.CompilerParams(
            dimension_semantics=("arbitrary",),
        ),
    )(te, valid, pos, x_bf, W1, W2, b1.reshape(E, 1, H),
      b2.reshape(E, 1, NA), ph2f)


def kernel(obs, expert_ids, ph_to_feature, W1, b1, W2, b2):
    x_bf = obs.reshape(N, D).astype(jnp.bfloat16)
    eid = expert_ids.reshape(-1).astype(jnp.int32)

    onehot = (eid[:, None] == jnp.arange(E, dtype=jnp.int32)[None, :])
    onehot = onehot.astype(jnp.int32)
    counts = jnp.sum(onehot, axis=0)
    rank = jnp.take_along_axis(jnp.cumsum(onehot, axis=0) - onehot,
                               eid[:, None], axis=1)[:, 0]
    tiles_per_e = (counts + TM - 1) // TM
    ctiles = jnp.cumsum(tiles_per_e)
    tile_start_e = ctiles - tiles_per_e
    pos = tile_start_e[eid] * TM + rank
    total_tiles = ctiles[E - 1]
    t_arr = jnp.arange(G, dtype=jnp.int32)
    te_raw = jnp.searchsorted(ctiles, t_arr, side="right").astype(jnp.int32)
    valid = (t_arr < total_tiles).astype(jnp.int32)
    last_e = jnp.searchsorted(ctiles, total_tiles - 1,
                              side="right").astype(jnp.int32)
    te = jnp.where(valid == 1, jnp.minimum(te_raw, E - 1), last_e)

    out_sorted = _tc_expert_tiles(te, valid, pos.reshape(1, N), x_bf,
                                  W1.astype(jnp.bfloat16),
                                  W2.astype(jnp.bfloat16),
                                  b1, b2, ph_to_feature)

    logp = _sc_gather_rows(out_sorted, pos.astype(jnp.int32),
                           rows_per_worker=N // NW, chunk=N // NW)
    return logp[:, :NA].reshape(T, A, NA)
